# Initial kernel scaffold; baseline (speedup 1.0000x reference)
import jax
import jax.numpy as jnp
from jax.experimental import pallas as pl


def kernel(x, edge_index, params):
    raise NotImplementedError("write your pallas kernel here")


# speedup vs baseline: 3.5608x; 3.5608x over previous
import functools

import jax
import jax.numpy as jnp
from jax import lax
from jax.experimental import pallas as pl
from jax.experimental.pallas import tpu as pltpu
from jax.experimental.pallas import tpu_sc as plsc

N_NODES = 10000
D = 128
OUT_DIM = 32
NUM_LAYERS = 3
BN_EPS = 1e-5

NC = 2
NS = 16
NW = NC * NS

NPAD = 10240
ROWS_PER_TILE = NPAD // NS

E_CHUNK = 128
N_EDGES = 320000
EPW_CHUNKS = -(-N_EDGES // (NW * E_CHUNK))
EPW = EPW_CHUNKS * E_CHUNK
EPAD = EPW * NW

_sc_mesh = plsc.VectorSubcoreMesh(core_axis_name="c", subcore_axis_name="s")


@functools.partial(
    pl.kernel,
    mesh=_sc_mesh,
    out_type=jax.ShapeDtypeStruct((NC, NPAD, D), jnp.float32),
    scratch_types=[
        pltpu.VMEM((1, E_CHUNK), jnp.int32),
        pltpu.VMEM((1, E_CHUNK), jnp.int32),
        pltpu.VMEM((E_CHUNK, D), jnp.float32),
        pltpu.VMEM_SHARED((NPAD, D), jnp.float32),
        pltpu.SemaphoreType.DMA,
    ],
)
def _sc_segment_sum(src_hbm, dst_hbm, h_hbm, zeros_hbm, out_hbm,
                    src_v, dst_v, rows_v, acc_sh, sem):
    cid = lax.axis_index("c")
    sid = lax.axis_index("s")
    wid = sid * NC + cid
    edge_base = wid * EPW
    row_base = sid * ROWS_PER_TILE

    pltpu.sync_copy(zeros_hbm, acc_sh.at[pl.ds(row_base, ROWS_PER_TILE)])
    plsc.subcore_barrier()

    def body(j, carry):
        off = edge_base + j * E_CHUNK
        pltpu.sync_copy(src_hbm.at[pl.ds(off, E_CHUNK)], src_v.at[0])
        pltpu.sync_copy(dst_hbm.at[pl.ds(off, E_CHUNK)], dst_v.at[0])
        pltpu.async_copy(h_hbm.at[src_v.at[0]], rows_v, sem).wait()
        pltpu.sync_copy(rows_v, acc_sh.at[dst_v.at[0]], add=True)
        return carry

    lax.fori_loop(0, EPW_CHUNKS, body, 0)

    plsc.subcore_barrier()
    pltpu.sync_copy(acc_sh.at[pl.ds(row_base, ROWS_PER_TILE)],
                    out_hbm.at[cid, pl.ds(row_base, ROWS_PER_TILE)])


def _mlp_body(h_ref, p0_ref, p1_ref, w1_ref, b1_ref, w2_ref, b2_ref,
              sc_ref, sh_ref, out_ref):
    agg = h_ref[...] + p0_ref[...] + p1_ref[...]
    h1 = jnp.maximum(
        jnp.dot(agg, w1_ref[...], preferred_element_type=jnp.float32)
        + b1_ref[...], 0.0)
    h2 = (jnp.dot(h1, w2_ref[...], preferred_element_type=jnp.float32)
          + b2_ref[...])
    out_ref[...] = jnp.maximum(h2 * sc_ref[...] + sh_ref[...], 0.0)


def _mlp_final_body(h_ref, p0_ref, p1_ref, w1_ref, b1_ref, w2_ref, b2_ref,
                    sc_ref, sh_ref, wc_ref, bc_ref, out_ref, cls_ref):
    _mlp_body(h_ref, p0_ref, p1_ref, w1_ref, b1_ref, w2_ref, b2_ref,
              sc_ref, sh_ref, out_ref)
    cls_ref[...] = (jnp.dot(out_ref[...], wc_ref[...],
                            preferred_element_type=jnp.float32) + bc_ref[...])


_BLK = 1024
_row_spec = pl.BlockSpec((_BLK, D), lambda i: (i, 0))
_w_spec = pl.BlockSpec((D, D), lambda i: (0, 0))
_v_spec = pl.BlockSpec((1, D), lambda i: (0, 0))


def _tc_mlp(h, p0, p1, w1, b1, w2, b2, scale, shift):
    return pl.pallas_call(
        _mlp_body,
        grid=(NPAD // _BLK,),
        in_specs=[_row_spec, _row_spec, _row_spec, _w_spec, _v_spec,
                  _w_spec, _v_spec, _v_spec, _v_spec],
        out_specs=_row_spec,
        out_shape=jax.ShapeDtypeStruct((NPAD, D), jnp.float32),
    )(h, p0, p1, w1, b1, w2, b2, scale, shift)


def _tc_mlp_final(h, p0, p1, w1, b1, w2, b2, scale, shift, wc, bc):
    return pl.pallas_call(
        _mlp_final_body,
        grid=(NPAD // _BLK,),
        in_specs=[_row_spec, _row_spec, _row_spec, _w_spec, _v_spec,
                  _w_spec, _v_spec, _v_spec, _v_spec, _w_spec, _v_spec],
        out_specs=(_row_spec, _row_spec),
        out_shape=(jax.ShapeDtypeStruct((NPAD, D), jnp.float32),
                   jax.ShapeDtypeStruct((NPAD, D), jnp.float32)),
    )(h, p0, p1, w1, b1, w2, b2, scale, shift, wc, bc)


def kernel(x, edge_index, params):
    ei = edge_index.astype(jnp.int32)
    pad_e = EPAD - N_EDGES
    src = jnp.concatenate(
        [ei[0], jnp.full((pad_e,), N_NODES, dtype=jnp.int32)])
    dst = jnp.concatenate(
        [ei[1], jnp.full((pad_e,), N_NODES, dtype=jnp.int32)])

    h = jnp.zeros((NPAD, D), jnp.float32).at[:N_NODES].set(x)
    zeros = jnp.zeros((ROWS_PER_TILE, D), jnp.float32)

    for i in range(NUM_LAYERS):
        cp = params[f'conv{i}']
        bn = params[f'bn{i}']
        scale = (bn['gamma'] * lax.rsqrt(bn['var'] + BN_EPS)).reshape(1, D)
        shift = (bn['beta'] - bn['mean'] * scale[0]).reshape(1, D)
        b1 = cp['b1'].reshape(1, D)
        b2 = cp['b2'].reshape(1, D)

        parts = _sc_segment_sum(src, dst, h, zeros)
        if i < NUM_LAYERS - 1:
            h = _tc_mlp(h, parts[0], parts[1], cp['W1'], b1,
                        cp['W2'], b2, scale, shift)
        else:
            wc = jnp.zeros((D, D), jnp.float32).at[:, :OUT_DIM].set(
                params['Wc'])
            bc = jnp.zeros((1, D), jnp.float32).at[0, :OUT_DIM].set(
                params['bc'])
            h, cls = _tc_mlp_final(h, parts[0], parts[1], cp['W1'], b1,
                                   cp['W2'], b2, scale, shift, wc, bc)
    return cls[:N_NODES, :OUT_DIM]
